# baseline (speedup 1.0000x reference)
import functools

import jax
import jax.numpy as jnp
from jax import lax
from jax.experimental import pallas as pl
from jax.experimental.pallas import tpu as pltpu
from jax.experimental.pallas import tpu_sc as plsc

N_NODES = 10000
N_PAD = 10112
D = 128
DD = 16
E = 320000
NC, NS = 2, 16
NW = NC * NS
CHUNK = 128
NCHUNK = 80
E_PAD = NW * NCHUNK * CHUNK
NBUF = 2
NIB = 4
ROWS_PER_TILE = N_PAD // NS


def _sc_aggregate(x, src_r, dst_r, zeros_a, zeros_d, ones_r):
    mesh = plsc.VectorSubcoreMesh(core_axis_name="c", subcore_axis_name="s")

    @functools.partial(
        pl.kernel,
        out_type=(jax.ShapeDtypeStruct((NC, N_PAD, D), jnp.float32),
                  jax.ShapeDtypeStruct((NC, N_PAD, DD), jnp.float32)),
        mesh=mesh,
        compiler_params=pltpu.CompilerParams(use_tc_tiling_on_sc=False),
        scratch_types=[
            pltpu.VMEM_SHARED((N_PAD, D), jnp.float32),
            pltpu.VMEM_SHARED((N_PAD, DD), jnp.float32),
            pltpu.VMEM((CHUNK, DD), jnp.float32),
        ]
        + [pltpu.VMEM((CHUNK,), jnp.int32) for _ in range(NIB)]
        + [pltpu.VMEM((CHUNK,), jnp.int32) for _ in range(NIB)]
        + [pltpu.VMEM((CHUNK, D), jnp.float32) for _ in range(NBUF)]
        + [pltpu.SemaphoreType.DMA for _ in range(2 * NIB + NBUF + 1)],
    )
    def k(x_hbm, src_hbm, dst_hbm, za_hbm, zd_hbm, ones_hbm,
          agg_hbm, deg_hbm, acc, dacc, ones_v, *bufs_sems):
        sbuf = bufs_sems[:NIB]
        dbuf = bufs_sems[NIB:2 * NIB]
        rows = bufs_sems[2 * NIB:2 * NIB + NBUF]
        isem = bufs_sems[2 * NIB + NBUF:3 * NIB + NBUF]
        dsem = bufs_sems[3 * NIB + NBUF:4 * NIB + NBUF]
        gsem = bufs_sems[4 * NIB + NBUF:4 * NIB + 2 * NBUF]
        osem = bufs_sems[-1]
        c = lax.axis_index("c")
        s = lax.axis_index("s")
        w = s * NC + c
        row0 = w * NCHUNK
        for t in range(NIB):
            pltpu.async_copy(src_hbm.at[row0 + t], sbuf[t], isem[t])
            pltpu.async_copy(dst_hbm.at[row0 + t], dbuf[t], dsem[t])
        pltpu.sync_copy(za_hbm, acc.at[pl.ds(s * ROWS_PER_TILE, ROWS_PER_TILE)])
        pltpu.sync_copy(zd_hbm, dacc.at[pl.ds(s * ROWS_PER_TILE, ROWS_PER_TILE)])
        pltpu.sync_copy(ones_hbm, ones_v)
        for t in range(NBUF):
            pltpu.make_async_copy(src_hbm.at[row0 + t], sbuf[t], isem[t]).wait()
            pltpu.async_copy(x_hbm.at[sbuf[t]], rows[t], gsem[t])
        plsc.subcore_barrier()

        def chunk_step(j, b, fire_idx, fire_gather):
            rb = b % NBUF
            pltpu.make_async_copy(dst_hbm.at[row0 + j], dbuf[b], dsem[b]).wait()
            pltpu.make_async_copy(x_hbm.at[sbuf[b]], rows[rb], gsem[rb]).wait()
            pltpu.sync_copy(rows[rb], acc.at[dbuf[b]], add=True)
            pltpu.sync_copy(ones_v, dacc.at[dbuf[b]], add=True)
            if fire_idx:
                pltpu.async_copy(src_hbm.at[row0 + j + NIB], sbuf[b], isem[b])
                pltpu.async_copy(dst_hbm.at[row0 + j + NIB], dbuf[b], dsem[b])
            if fire_gather:
                b2 = (b + NBUF) % NIB
                pltpu.make_async_copy(src_hbm.at[row0 + j + NBUF], sbuf[b2], isem[b2]).wait()
                pltpu.async_copy(x_hbm.at[sbuf[b2]], rows[rb], gsem[rb])

        def ring(g, carry):
            for b in range(NIB):
                chunk_step(g * NIB + b, b, True, True)
            return carry

        main_end = ((NCHUNK - NIB) // NIB) * NIB
        lax.fori_loop(0, main_end // NIB, ring, 0)
        for j in range(main_end, NCHUNK):
            chunk_step(j, j % NIB, j + NIB < NCHUNK, j + NBUF < NCHUNK)

        plsc.subcore_barrier()
        pltpu.async_copy(
            acc.at[pl.ds(s * ROWS_PER_TILE, ROWS_PER_TILE)],
            agg_hbm.at[c, pl.ds(s * ROWS_PER_TILE, ROWS_PER_TILE)],
            osem,
        )
        pltpu.sync_copy(
            dacc.at[pl.ds(s * ROWS_PER_TILE, ROWS_PER_TILE)],
            deg_hbm.at[c, pl.ds(s * ROWS_PER_TILE, ROWS_PER_TILE)],
        )
        pltpu.make_async_copy(
            acc.at[pl.ds(s * ROWS_PER_TILE, ROWS_PER_TILE)],
            agg_hbm.at[c, pl.ds(s * ROWS_PER_TILE, ROWS_PER_TILE)],
            osem,
        ).wait()

    return k(x, src_r, dst_r, zeros_a, zeros_d, ones_r)


BLK = 1000


def _tc_tail(agg, deg, x, W_l, W_r, b_l2d):
    def body(a0_ref, a1_ref, d0_ref, d1_ref, x_ref, wl_ref, wr_ref, b_ref, o_ref):
        p = a0_ref[0] + a1_ref[0]
        d = d0_ref[0][:, 0:1] + d1_ref[0][:, 0:1]
        mean = p / jnp.maximum(d, 1.0)
        h = (jnp.dot(mean, wl_ref[...], preferred_element_type=jnp.float32)
             + b_ref[...]
             + jnp.dot(x_ref[...], wr_ref[...], preferred_element_type=jnp.float32))
        h = jnp.maximum(h, 0.0)
        n = jnp.sqrt(jnp.sum(h * h, axis=1, keepdims=True))
        o_ref[...] = h / (n + 1e-9)

    return pl.pallas_call(
        body,
        grid=(N_NODES // BLK,),
        in_specs=[
            pl.BlockSpec((1, BLK, D), lambda i: (0, i, 0)),
            pl.BlockSpec((1, BLK, D), lambda i: (1, i, 0)),
            pl.BlockSpec((1, BLK, DD), lambda i: (0, i, 0)),
            pl.BlockSpec((1, BLK, DD), lambda i: (1, i, 0)),
            pl.BlockSpec((BLK, D), lambda i: (i, 0)),
            pl.BlockSpec((D, D), lambda i: (0, 0)),
            pl.BlockSpec((D, D), lambda i: (0, 0)),
            pl.BlockSpec((1, D), lambda i: (0, 0)),
        ],
        out_specs=pl.BlockSpec((BLK, D), lambda i: (i, 0)),
        out_shape=jax.ShapeDtypeStruct((N_NODES, D), jnp.float32),
    )(agg, agg, deg, deg, x, W_l, W_r, b_l2d)


def kernel(x, edge_index, W_l, W_r, b_l):
    ei = edge_index.astype(jnp.int32)
    npad = E_PAD - E
    src_r = jnp.concatenate(
        [ei[0], jnp.zeros((npad,), jnp.int32)]).reshape(NW * NCHUNK, CHUNK)
    dst_r = jnp.concatenate(
        [ei[1],
         (jnp.arange(npad, dtype=jnp.int32) % (N_PAD - N_NODES)) + N_NODES]
    ).reshape(NW * NCHUNK, CHUNK)
    zeros_a = jnp.zeros((ROWS_PER_TILE, D), jnp.float32)
    zeros_d = jnp.zeros((ROWS_PER_TILE, DD), jnp.float32)
    ones_r = jnp.ones((CHUNK, DD), jnp.float32)
    agg, deg = _sc_aggregate(x, src_r, dst_r, zeros_a, zeros_d, ones_r)
    return _tc_tail(agg, deg, x, W_l, W_r, b_l.reshape(1, D))


# speedup vs baseline: 2.6116x; 2.6116x over previous
import functools

import jax
import jax.numpy as jnp
from jax import lax
from jax.experimental import pallas as pl
from jax.experimental.pallas import tpu as pltpu
from jax.experimental.pallas import tpu_sc as plsc

N_NODES = 10000
N_PAD = 10240
D = 128
DD = 16
E = 320000
NC, NS = 2, 16
NW = NC * NS
CHUNK = 128
NROWS = E // CHUNK
NCHUNK = NROWS // NW
NXTRA = NROWS - NW * NCHUNK
NBUF = 2
NIB = 4
ROWS_PER_TILE = N_PAD // NS


def _sc_aggregate(x, src_r, dst_r, zeros_a, zeros_d, ones_r):
    mesh = plsc.VectorSubcoreMesh(core_axis_name="c", subcore_axis_name="s")

    @functools.partial(
        pl.kernel,
        out_type=(jax.ShapeDtypeStruct((NC, N_PAD, D), jnp.float32),
                  jax.ShapeDtypeStruct((NC, N_PAD, DD), jnp.float32)),
        mesh=mesh,
        compiler_params=pltpu.CompilerParams(use_tc_tiling_on_sc=False),
        scratch_types=[
            pltpu.VMEM_SHARED((N_PAD, D), jnp.float32),
            pltpu.VMEM_SHARED((N_PAD, DD), jnp.float32),
            pltpu.VMEM((CHUNK, DD), jnp.float32),
        ]
        + [pltpu.VMEM((CHUNK,), jnp.int32) for _ in range(NIB)]
        + [pltpu.VMEM((CHUNK,), jnp.int32) for _ in range(NIB)]
        + [pltpu.VMEM((CHUNK, D), jnp.float32) for _ in range(NBUF)]
        + [pltpu.SemaphoreType.DMA for _ in range(2 * NIB + NBUF + 1)],
    )
    def k(x_hbm, src_hbm, dst_hbm, za_hbm, zd_hbm, ones_hbm,
          agg_hbm, deg_hbm, acc, dacc, ones_v, *bufs_sems):
        sbuf = bufs_sems[:NIB]
        dbuf = bufs_sems[NIB:2 * NIB]
        rows = bufs_sems[2 * NIB:2 * NIB + NBUF]
        isem = bufs_sems[2 * NIB + NBUF:3 * NIB + NBUF]
        dsem = bufs_sems[3 * NIB + NBUF:4 * NIB + NBUF]
        gsem = bufs_sems[4 * NIB + NBUF:4 * NIB + 2 * NBUF]
        osem = bufs_sems[-1]
        c = lax.axis_index("c")
        s = lax.axis_index("s")
        w = s * NC + c
        last = NROWS - 1

        def erow(j, clamp=False):
            r = w + NW * j
            return jnp.minimum(r, last) if clamp else r

        for t in range(NIB):
            pltpu.async_copy(src_hbm.at[erow(t)], sbuf[t], isem[t])
            pltpu.async_copy(dst_hbm.at[erow(t)], dbuf[t], dsem[t])
        pltpu.sync_copy(za_hbm, acc.at[pl.ds(s * ROWS_PER_TILE, ROWS_PER_TILE)])
        pltpu.sync_copy(zd_hbm, dacc.at[pl.ds(s * ROWS_PER_TILE, ROWS_PER_TILE)])
        pltpu.sync_copy(ones_hbm, ones_v)
        for t in range(NBUF):
            pltpu.make_async_copy(src_hbm.at[erow(t)], sbuf[t], isem[t]).wait()
            pltpu.async_copy(x_hbm.at[sbuf[t]], rows[t], gsem[t])
        plsc.subcore_barrier()

        def chunk_step(j, b, fire_idx, fire_gather, clamp=False):
            rb = b % NBUF
            pltpu.make_async_copy(dst_hbm.at[erow(j)], dbuf[b], dsem[b]).wait()
            pltpu.make_async_copy(x_hbm.at[sbuf[b]], rows[rb], gsem[rb]).wait()
            pltpu.sync_copy(rows[rb], acc.at[dbuf[b]], add=True)
            pltpu.sync_copy(ones_v, dacc.at[dbuf[b]], add=True)
            if fire_idx:
                r = erow(j + NIB, clamp)
                pltpu.async_copy(src_hbm.at[r], sbuf[b], isem[b])
                pltpu.async_copy(dst_hbm.at[r], dbuf[b], dsem[b])
            if fire_gather:
                b2 = (b + NBUF) % NIB
                pltpu.make_async_copy(src_hbm.at[erow(j + NBUF, clamp)],
                                      sbuf[b2], isem[b2]).wait()
                pltpu.async_copy(x_hbm.at[sbuf[b2]], rows[rb], gsem[rb])

        def ring(g, carry):
            for b in range(NIB):
                chunk_step(g * NIB + b, b, True, True)
            return carry

        main_end = ((NCHUNK - NIB) // NIB) * NIB
        lax.fori_loop(0, main_end // NIB, ring, 0)
        for j in range(main_end, NCHUNK):
            chunk_step(j, j % NIB, j + NIB <= NCHUNK, j + NBUF <= NCHUNK,
                       clamp=True)

        bx = NCHUNK % NIB
        rbx = bx % NBUF
        pltpu.make_async_copy(dst_hbm.at[erow(NCHUNK, True)], dbuf[bx],
                              dsem[bx]).wait()
        pltpu.make_async_copy(x_hbm.at[sbuf[bx]], rows[rbx], gsem[rbx]).wait()

        @pl.when(w >= NXTRA)
        def _():
            for kk in range(CHUNK // 16):
                dbuf[bx][pl.ds(kk * 16, 16)] = (
                    N_NODES + kk * 16 + lax.iota(jnp.int32, 16))

        pltpu.sync_copy(rows[rbx], acc.at[dbuf[bx]], add=True)
        pltpu.sync_copy(ones_v, dacc.at[dbuf[bx]], add=True)

        plsc.subcore_barrier()
        pltpu.async_copy(
            acc.at[pl.ds(s * ROWS_PER_TILE, ROWS_PER_TILE)],
            agg_hbm.at[c, pl.ds(s * ROWS_PER_TILE, ROWS_PER_TILE)],
            osem,
        )
        pltpu.sync_copy(
            dacc.at[pl.ds(s * ROWS_PER_TILE, ROWS_PER_TILE)],
            deg_hbm.at[c, pl.ds(s * ROWS_PER_TILE, ROWS_PER_TILE)],
        )
        pltpu.make_async_copy(
            acc.at[pl.ds(s * ROWS_PER_TILE, ROWS_PER_TILE)],
            agg_hbm.at[c, pl.ds(s * ROWS_PER_TILE, ROWS_PER_TILE)],
            osem,
        ).wait()

    return k(x, src_r, dst_r, zeros_a, zeros_d, ones_r)


BLK = 1000


def _tc_tail(agg, deg, x, W_l, W_r, b_l2d):
    def body(a0_ref, a1_ref, d0_ref, d1_ref, x_ref, wl_ref, wr_ref, b_ref, o_ref):
        p = a0_ref[0] + a1_ref[0]
        d = d0_ref[0][:, 0:1] + d1_ref[0][:, 0:1]
        mean = p / jnp.maximum(d, 1.0)
        h = (jnp.dot(mean, wl_ref[...], preferred_element_type=jnp.float32)
             + b_ref[...]
             + jnp.dot(x_ref[...], wr_ref[...], preferred_element_type=jnp.float32))
        h = jnp.maximum(h, 0.0)
        n = jnp.sqrt(jnp.sum(h * h, axis=1, keepdims=True))
        o_ref[...] = h / (n + 1e-9)

    return pl.pallas_call(
        body,
        grid=(N_NODES // BLK,),
        in_specs=[
            pl.BlockSpec((1, BLK, D), lambda i: (0, i, 0)),
            pl.BlockSpec((1, BLK, D), lambda i: (1, i, 0)),
            pl.BlockSpec((1, BLK, DD), lambda i: (0, i, 0)),
            pl.BlockSpec((1, BLK, DD), lambda i: (1, i, 0)),
            pl.BlockSpec((BLK, D), lambda i: (i, 0)),
            pl.BlockSpec((D, D), lambda i: (0, 0)),
            pl.BlockSpec((D, D), lambda i: (0, 0)),
            pl.BlockSpec((1, D), lambda i: (0, 0)),
        ],
        out_specs=pl.BlockSpec((BLK, D), lambda i: (i, 0)),
        out_shape=jax.ShapeDtypeStruct((N_NODES, D), jnp.float32),
    )(agg, agg, deg, deg, x, W_l, W_r, b_l2d)


def kernel(x, edge_index, W_l, W_r, b_l):
    ei = edge_index.astype(jnp.int32)
    src_r = ei[0].reshape(NROWS, CHUNK)
    dst_r = ei[1].reshape(NROWS, CHUNK)
    zeros_a = jnp.zeros((ROWS_PER_TILE, D), jnp.float32)
    zeros_d = jnp.zeros((ROWS_PER_TILE, DD), jnp.float32)
    ones_r = jnp.ones((CHUNK, DD), jnp.float32)
    agg, deg = _sc_aggregate(x, src_r, dst_r, zeros_a, zeros_d, ones_r)
    return _tc_tail(agg, deg, x, W_l, W_r, b_l.reshape(1, D))


# speedup vs baseline: 2.6883x; 1.0293x over previous
import functools

import jax
import jax.numpy as jnp
from jax import lax
from jax.experimental import pallas as pl
from jax.experimental.pallas import tpu as pltpu
from jax.experimental.pallas import tpu_sc as plsc

N_NODES = 10000
N_PAD = 10240
D = 128
DD = 16
E = 320000
NC, NS = 2, 16
NW = NC * NS
CHUNK = 128
NROWS = E // CHUNK
NCHUNK = NROWS // NW
NXTRA = NROWS - NW * NCHUNK
NBUF = 2
NIB = 4
ROWS_PER_TILE = N_PAD // NS


def _sc_aggregate(x, src_r, dst_r, zeros_a, zeros_d, ones_r):
    mesh = plsc.VectorSubcoreMesh(core_axis_name="c", subcore_axis_name="s")

    @functools.partial(
        pl.kernel,
        out_type=(jax.ShapeDtypeStruct((NC, N_PAD, D), jnp.float32),
                  jax.ShapeDtypeStruct((NC, N_PAD, DD), jnp.int16)),
        mesh=mesh,
        compiler_params=pltpu.CompilerParams(use_tc_tiling_on_sc=False),
        scratch_types=[
            pltpu.VMEM_SHARED((N_PAD, D), jnp.float32),
            pltpu.VMEM_SHARED((N_PAD, DD), jnp.int16),
            pltpu.VMEM((CHUNK, DD), jnp.int16),
        ]
        + [pltpu.VMEM((CHUNK,), jnp.int32) for _ in range(NIB)]
        + [pltpu.VMEM((CHUNK,), jnp.int32) for _ in range(NIB)]
        + [pltpu.VMEM((CHUNK, D), jnp.float32) for _ in range(NBUF)]
        + [pltpu.SemaphoreType.DMA for _ in range(2 * NIB + NBUF + 1)],
    )
    def k(x_hbm, src_hbm, dst_hbm, za_hbm, zd_hbm, ones_hbm,
          agg_hbm, deg_hbm, acc, dacc, ones_v, *bufs_sems):
        sbuf = bufs_sems[:NIB]
        dbuf = bufs_sems[NIB:2 * NIB]
        rows = bufs_sems[2 * NIB:2 * NIB + NBUF]
        isem = bufs_sems[2 * NIB + NBUF:3 * NIB + NBUF]
        dsem = bufs_sems[3 * NIB + NBUF:4 * NIB + NBUF]
        gsem = bufs_sems[4 * NIB + NBUF:4 * NIB + 2 * NBUF]
        osem = bufs_sems[-1]
        c = lax.axis_index("c")
        s = lax.axis_index("s")
        w = s * NC + c
        last = NROWS - 1

        def erow(j, clamp=False):
            r = w + NW * j
            return jnp.minimum(r, last) if clamp else r

        for t in range(NIB):
            pltpu.async_copy(src_hbm.at[erow(t)], sbuf[t], isem[t])
            pltpu.async_copy(dst_hbm.at[erow(t)], dbuf[t], dsem[t])
        pltpu.sync_copy(za_hbm, acc.at[pl.ds(s * ROWS_PER_TILE, ROWS_PER_TILE)])
        pltpu.sync_copy(zd_hbm, dacc.at[pl.ds(s * ROWS_PER_TILE, ROWS_PER_TILE)])
        pltpu.sync_copy(ones_hbm, ones_v)
        for t in range(NBUF):
            pltpu.make_async_copy(src_hbm.at[erow(t)], sbuf[t], isem[t]).wait()
            pltpu.async_copy(x_hbm.at[sbuf[t]], rows[t], gsem[t])
        plsc.subcore_barrier()

        def chunk_step(j, b, fire_idx, fire_gather, clamp=False):
            rb = b % NBUF
            pltpu.make_async_copy(dst_hbm.at[erow(j)], dbuf[b], dsem[b]).wait()
            pltpu.make_async_copy(x_hbm.at[sbuf[b]], rows[rb], gsem[rb]).wait()
            pltpu.sync_copy(rows[rb], acc.at[dbuf[b]], add=True)
            pltpu.sync_copy(ones_v, dacc.at[dbuf[b]], add=True)
            if fire_idx:
                r = erow(j + NIB, clamp)
                pltpu.async_copy(src_hbm.at[r], sbuf[b], isem[b])
                pltpu.async_copy(dst_hbm.at[r], dbuf[b], dsem[b])
            if fire_gather:
                b2 = (b + NBUF) % NIB
                pltpu.make_async_copy(src_hbm.at[erow(j + NBUF, clamp)],
                                      sbuf[b2], isem[b2]).wait()
                pltpu.async_copy(x_hbm.at[sbuf[b2]], rows[rb], gsem[rb])

        def ring(g, carry):
            for b in range(NIB):
                chunk_step(g * NIB + b, b, True, True)
            return carry

        main_end = ((NCHUNK - NIB) // NIB) * NIB
        lax.fori_loop(0, main_end // NIB, ring, 0)
        for j in range(main_end, NCHUNK):
            chunk_step(j, j % NIB, j + NIB <= NCHUNK, j + NBUF <= NCHUNK,
                       clamp=True)

        bx = NCHUNK % NIB
        rbx = bx % NBUF
        pltpu.make_async_copy(dst_hbm.at[erow(NCHUNK, True)], dbuf[bx],
                              dsem[bx]).wait()
        pltpu.make_async_copy(x_hbm.at[sbuf[bx]], rows[rbx], gsem[rbx]).wait()

        @pl.when(w >= NXTRA)
        def _():
            for kk in range(CHUNK // 16):
                dbuf[bx][pl.ds(kk * 16, 16)] = (
                    N_NODES + kk * 16 + lax.iota(jnp.int32, 16))

        pltpu.sync_copy(rows[rbx], acc.at[dbuf[bx]], add=True)
        pltpu.sync_copy(ones_v, dacc.at[dbuf[bx]], add=True)

        plsc.subcore_barrier()
        pltpu.async_copy(
            acc.at[pl.ds(s * ROWS_PER_TILE, ROWS_PER_TILE)],
            agg_hbm.at[c, pl.ds(s * ROWS_PER_TILE, ROWS_PER_TILE)],
            osem,
        )
        pltpu.sync_copy(
            dacc.at[pl.ds(s * ROWS_PER_TILE, ROWS_PER_TILE)],
            deg_hbm.at[c, pl.ds(s * ROWS_PER_TILE, ROWS_PER_TILE)],
        )
        pltpu.make_async_copy(
            acc.at[pl.ds(s * ROWS_PER_TILE, ROWS_PER_TILE)],
            agg_hbm.at[c, pl.ds(s * ROWS_PER_TILE, ROWS_PER_TILE)],
            osem,
        ).wait()

    return k(x, src_r, dst_r, zeros_a, zeros_d, ones_r)


BLK = 2000


def _tc_tail(agg, deg, x, W_l, W_r, b_l2d):
    def body(a0_ref, a1_ref, d0_ref, d1_ref, x_ref, wl_ref, wr_ref, b_ref, o_ref):
        p = a0_ref[0] + a1_ref[0]
        d = (d0_ref[0][:, 0:1] + d1_ref[0][:, 0:1]).astype(jnp.float32)
        mean = p / jnp.maximum(d, 1.0)
        h = (jnp.dot(mean, wl_ref[...], preferred_element_type=jnp.float32)
             + b_ref[...]
             + jnp.dot(x_ref[...], wr_ref[...], preferred_element_type=jnp.float32))
        h = jnp.maximum(h, 0.0)
        n = jnp.sqrt(jnp.sum(h * h, axis=1, keepdims=True))
        o_ref[...] = h / (n + 1e-9)

    return pl.pallas_call(
        body,
        grid=(N_NODES // BLK,),
        in_specs=[
            pl.BlockSpec((1, BLK, D), lambda i: (0, i, 0)),
            pl.BlockSpec((1, BLK, D), lambda i: (1, i, 0)),
            pl.BlockSpec((1, BLK, DD), lambda i: (0, i, 0)),
            pl.BlockSpec((1, BLK, DD), lambda i: (1, i, 0)),
            pl.BlockSpec((BLK, D), lambda i: (i, 0)),
            pl.BlockSpec((D, D), lambda i: (0, 0)),
            pl.BlockSpec((D, D), lambda i: (0, 0)),
            pl.BlockSpec((1, D), lambda i: (0, 0)),
        ],
        out_specs=pl.BlockSpec((BLK, D), lambda i: (i, 0)),
        out_shape=jax.ShapeDtypeStruct((N_NODES, D), jnp.float32),
    )(agg, agg, deg, deg, x, W_l, W_r, b_l2d)


def kernel(x, edge_index, W_l, W_r, b_l):
    ei = edge_index.astype(jnp.int32)
    src_r = ei[0].reshape(NROWS, CHUNK)
    dst_r = ei[1].reshape(NROWS, CHUNK)
    zeros_a = jnp.zeros((ROWS_PER_TILE, D), jnp.float32)
    zeros_d = jnp.zeros((ROWS_PER_TILE, DD), jnp.int16)
    ones_r = jnp.ones((CHUNK, DD), jnp.int16)
    agg, deg = _sc_aggregate(x, src_r, dst_r, zeros_a, zeros_d, ones_r)
    return _tc_tail(agg, deg, x, W_l, W_r, b_l.reshape(1, D))
